# baseline (speedup 1.0000x reference)
import functools

import jax
import jax.numpy as jnp
from jax import lax
from jax.experimental import pallas as pl
from jax.experimental.pallas import tpu as pltpu
from jax.experimental.pallas import tpu_sc as plsc


def _make_sc_gather(N, DW):
    info = plsc.get_sparse_core_info()
    NC, NS = info.num_cores, info.num_subcores
    NW = NC * NS
    assert N % NW == 0
    tok_w = N // NW
    CH = 128
    assert tok_w % (2 * CH) == 0
    n_chunks = tok_w // CH

    mesh = plsc.VectorSubcoreMesh(core_axis_name="c", subcore_axis_name="s")

    @functools.partial(
        pl.kernel,
        mesh=mesh,
        compiler_params=pltpu.CompilerParams(
            use_tc_tiling_on_sc=False, needs_layout_passes=False),
        out_type=jax.ShapeDtypeStruct((N * DW,), jnp.int32),
        scratch_types=[
            pltpu.VMEM((CH,), jnp.int32),
            pltpu.VMEM((CH,), jnp.int32),
            pltpu.VMEM((CH, DW), jnp.int32),
            pltpu.VMEM((CH, DW), jnp.int32),
            pltpu.VMEM((CH * DW,), jnp.int32),
            pltpu.VMEM((CH * DW,), jnp.int32),
            pltpu.SemaphoreType.DMA,
            pltpu.SemaphoreType.DMA,
            pltpu.SemaphoreType.DMA,
            pltpu.SemaphoreType.DMA,
            pltpu.SemaphoreType.DMA,
            pltpu.SemaphoreType.DMA,
        ],
    )
    def sc_gather(cat_h, wtab_h, out_h,
                  cat0, cat1, w0b, w1b, f0b, f1b,
                  sidx0, sidx1, srow0, srow1, sout0, sout1):
        wid = lax.axis_index("s") * NC + lax.axis_index("c")
        base = wid * tok_w
        slots = (
            (cat0, w0b, sidx0, srow0, sout0, f0b),
            (cat1, w1b, sidx1, srow1, sout1, f1b),
        )

        def issue_idx(g, sl):
            cat_v, sidx = sl[0], sl[2]
            tok0 = base + g * CH
            pltpu.async_copy(cat_h.at[pl.ds(tok0, CH)], cat_v, sidx)

        def wait_idx(sl):
            cat_v, sidx = sl[0], sl[2]
            pltpu.make_async_copy(cat_h.at[pl.ds(base, CH)], cat_v, sidx).wait()

        def issue_gather(sl):
            cat_v, w_v, srow = sl[0], sl[1], sl[3]
            pltpu.async_copy(wtab_h.at[cat_v], w_v, srow)

        def wait_gather(sl):
            cat_v, w_v, srow = sl[0], sl[1], sl[3]
            pltpu.make_async_copy(wtab_h.at[cat_v], w_v, srow).wait()

        def repack(sl):
            cat_v, w_v, f_v = sl[0], sl[1], sl[5]

            def fix_grp(i, c):
                ids = cat_v[pl.ds(i * 16, 16)]
                msk = ids == 0
                nbad = jnp.max(msk.astype(jnp.int32))

                @pl.when(nbad > 0)
                def _():
                    rows = lax.iota(jnp.int32, 16) + i * 16
                    zeros = jnp.zeros((16,), jnp.int32)
                    for col in range(DW):
                        plsc.store_scatter(
                            w_v, [rows, jnp.full((16,), col, jnp.int32)],
                            zeros, mask=msk)
                return c

            lax.fori_loop(0, CH // 16, fix_grp, 0)

            def tok_body(t, c):
                for j in range(DW // 16):
                    f_v[pl.ds(t * DW + j * 16, 16)] = w_v[t, pl.ds(j * 16, 16)]
                return c

            lax.fori_loop(0, CH, tok_body, 0)

        def issue_out(g, sl):
            f_v, sout = sl[5], sl[4]
            el0 = (base + g * CH) * DW
            pltpu.async_copy(f_v, out_h.at[pl.ds(el0, CH * DW)], sout)

        def wait_out(sl):
            f_v, sout = sl[5], sl[4]
            pltpu.make_async_copy(
                f_v, out_h.at[pl.ds(base * DW, CH * DW)], sout).wait()

        issue_idx(0, slots[0])
        issue_idx(1, slots[1])
        wait_idx(slots[0])
        issue_gather(slots[0])

        def outer(go, carry):
            for b in (0, 1):
                g = 2 * go + b
                cur, nxt = slots[b], slots[1 - b]

                @pl.when(g + 1 < n_chunks)
                def _():
                    wait_idx(nxt)

                    @pl.when(g >= 1)
                    def _():
                        wait_out(nxt)

                    issue_gather(nxt)

                wait_gather(cur)

                @pl.when(g + 2 < n_chunks)
                def _():
                    issue_idx(g + 2, cur)

                repack(cur)
                issue_out(g, cur)
            return carry

        lax.fori_loop(0, n_chunks // 2, outer, 0)
        wait_out(slots[0])
        wait_out(slots[1])

    return sc_gather


def _make_tc_assemble(N, D, DW, P, S, TB):
    NB = N // TB

    def body(w_ref, pid_ref, sid_ref, ptab_ref, stab_ref, o_ref):
        wi = w_ref[...]
        lo_f = lax.bitcast_convert_type(wi << 16, jnp.float32)
        hi_f = lax.bitcast_convert_type(
            wi & jnp.int32(-65536), jnp.float32)
        w = jnp.concatenate([lo_f, hi_f], axis=1)
        pid = pid_ref[0, 0, :]
        sid = sid_ref[0, 0, :]
        iota_p = lax.broadcasted_iota(jnp.int32, (TB, P), 1)
        iota_s = lax.broadcasted_iota(jnp.int32, (TB, S), 1)
        oh_p = (iota_p == pid[:, None]).astype(jnp.bfloat16)
        oh_s = (iota_s == sid[:, None]).astype(jnp.bfloat16)
        pp = jnp.dot(oh_p, ptab_ref[...], preferred_element_type=jnp.float32)
        ss = jnp.dot(oh_s, stab_ref[...], preferred_element_type=jnp.float32)
        o_ref[...] = w + pp + ss

    return pl.pallas_call(
        body,
        grid=(NB,),
        in_specs=[
            pl.BlockSpec((TB, DW), lambda i: (i, 0)),
            pl.BlockSpec((1, 1, TB), lambda i: (i, 0, 0)),
            pl.BlockSpec((1, 1, TB), lambda i: (i, 0, 0)),
            pl.BlockSpec((P, D), lambda i: (0, 0)),
            pl.BlockSpec((S, D), lambda i: (0, 0)),
        ],
        out_specs=pl.BlockSpec((TB, D), lambda i: (i, 0)),
        out_shape=jax.ShapeDtypeStruct((N, D), jnp.float32),
    )


def kernel(cat_ids, position_ids, shape_ids, word_table, pos_table, shape_table):
    B, L = cat_ids.shape
    V, D = word_table.shape
    P = pos_table.shape[0]
    S = shape_table.shape[0]
    N = B * L
    DW = D // 2
    TB = 4096

    wtab16 = word_table.astype(jnp.bfloat16)
    wtab_pairs = lax.bitcast_convert_type(
        jnp.stack([wtab16[:, :DW], wtab16[:, DW:]], axis=-1), jnp.int32)

    H = N // 2
    cat_flat = cat_ids.reshape(N)
    pid3 = position_ids.reshape(N // TB, 1, TB)
    sid3 = shape_ids.reshape(N // TB, 1, TB)
    HB = H // TB
    sck = _make_sc_gather(H, DW)
    tcb = _make_tc_assemble(H, D, DW, P, S, TB)
    ptab16 = pos_table.astype(jnp.bfloat16)
    stab16 = shape_table.astype(jnp.bfloat16)

    halves = []
    for h in range(2):
        wr = sck(cat_flat[h * H:(h + 1) * H], wtab_pairs)
        halves.append((wr, pid3[h * HB:(h + 1) * HB], sid3[h * HB:(h + 1) * HB]))
    outs = [
        tcb(wr.reshape(H, DW), pid, sid, ptab16, stab16)
        for wr, pid, sid in halves
    ]
    return jnp.concatenate(outs, axis=0).reshape(B, L, D)


# speedup vs baseline: 1.2312x; 1.2312x over previous
import functools

import jax
import jax.numpy as jnp
from jax import lax
from jax.experimental import pallas as pl
from jax.experimental.pallas import tpu as pltpu
from jax.experimental.pallas import tpu_sc as plsc


def _make_sc_gather(N, DW):
    info = plsc.get_sparse_core_info()
    NC, NS = info.num_cores, info.num_subcores
    NW = NC * NS
    assert N % NW == 0
    tok_w = N // NW
    CH = 128
    assert tok_w % (2 * CH) == 0
    n_chunks = tok_w // CH

    mesh = plsc.VectorSubcoreMesh(core_axis_name="c", subcore_axis_name="s")

    @functools.partial(
        pl.kernel,
        mesh=mesh,
        compiler_params=pltpu.CompilerParams(
            use_tc_tiling_on_sc=False, needs_layout_passes=False),
        out_type=jax.ShapeDtypeStruct((N * DW,), jnp.int32),
        scratch_types=[
            pltpu.VMEM((CH,), jnp.int32),
            pltpu.VMEM((CH,), jnp.int32),
            pltpu.VMEM((CH, DW), jnp.int32),
            pltpu.VMEM((CH, DW), jnp.int32),
            pltpu.VMEM((CH * DW,), jnp.int32),
            pltpu.VMEM((CH * DW,), jnp.int32),
            pltpu.SemaphoreType.DMA,
            pltpu.SemaphoreType.DMA,
            pltpu.SemaphoreType.DMA,
            pltpu.SemaphoreType.DMA,
            pltpu.SemaphoreType.DMA,
            pltpu.SemaphoreType.DMA,
        ],
    )
    def sc_gather(cat_h, wtab_h, out_h,
                  cat0, cat1, w0b, w1b, f0b, f1b,
                  sidx0, sidx1, srow0, srow1, sout0, sout1):
        wid = lax.axis_index("s") * NC + lax.axis_index("c")
        base = wid * tok_w
        slots = (
            (cat0, w0b, sidx0, srow0, sout0, f0b),
            (cat1, w1b, sidx1, srow1, sout1, f1b),
        )

        def issue_idx(g, sl):
            cat_v, sidx = sl[0], sl[2]
            tok0 = base + g * CH
            pltpu.async_copy(cat_h.at[pl.ds(tok0, CH)], cat_v, sidx)

        def wait_idx(sl):
            cat_v, sidx = sl[0], sl[2]
            pltpu.make_async_copy(cat_h.at[pl.ds(base, CH)], cat_v, sidx).wait()

        def issue_gather(sl):
            cat_v, w_v, srow = sl[0], sl[1], sl[3]
            pltpu.async_copy(wtab_h.at[cat_v], w_v, srow)

        def wait_gather(sl):
            cat_v, w_v, srow = sl[0], sl[1], sl[3]
            pltpu.make_async_copy(wtab_h.at[cat_v], w_v, srow).wait()

        def repack(sl):
            cat_v, w_v, f_v = sl[0], sl[1], sl[5]

            def fix_grp(i, c):
                ids = cat_v[pl.ds(i * 16, 16)]
                msk = ids == 0
                nbad = jnp.max(msk.astype(jnp.int32))

                @pl.when(nbad > 0)
                def _():
                    rows = lax.iota(jnp.int32, 16) + i * 16
                    zeros = jnp.zeros((16,), jnp.int32)
                    for col in range(DW):
                        plsc.store_scatter(
                            w_v, [rows, jnp.full((16,), col, jnp.int32)],
                            zeros, mask=msk)
                return c

            lax.fori_loop(0, CH // 16, fix_grp, 0)

            def tok_body(t, c):
                for j in range(DW // 16):
                    f_v[pl.ds(t * DW + j * 16, 16)] = w_v[t, pl.ds(j * 16, 16)]
                return c

            lax.fori_loop(0, CH, tok_body, 0)

        def issue_out(g, sl):
            f_v, sout = sl[5], sl[4]
            el0 = (base + g * CH) * DW
            pltpu.async_copy(f_v, out_h.at[pl.ds(el0, CH * DW)], sout)

        def wait_out(sl):
            f_v, sout = sl[5], sl[4]
            pltpu.make_async_copy(
                f_v, out_h.at[pl.ds(base * DW, CH * DW)], sout).wait()

        issue_idx(0, slots[0])
        issue_idx(1, slots[1])
        wait_idx(slots[0])
        issue_gather(slots[0])

        def outer(go, carry):
            for b in (0, 1):
                g = 2 * go + b
                cur, nxt = slots[b], slots[1 - b]

                @pl.when(g + 1 < n_chunks)
                def _():
                    wait_idx(nxt)

                    @pl.when(g >= 1)
                    def _():
                        wait_out(nxt)

                    issue_gather(nxt)

                wait_gather(cur)

                @pl.when(g + 2 < n_chunks)
                def _():
                    issue_idx(g + 2, cur)

                repack(cur)
                issue_out(g, cur)
            return carry

        lax.fori_loop(0, n_chunks // 2, outer, 0)
        wait_out(slots[0])
        wait_out(slots[1])

    return sc_gather


def _make_tc_assemble(N, D, DW, P, S, TB):
    NB = N // TB

    def body(w_ref, pid_ref, sid_ref, ptab_ref, stab_ref, o_ref):
        wi = w_ref[...]
        lo_f = lax.bitcast_convert_type(wi << 16, jnp.float32)
        hi_f = lax.bitcast_convert_type(
            wi & jnp.int32(-65536), jnp.float32)
        w = jnp.concatenate([lo_f, hi_f], axis=1)
        pid = pid_ref[0, 0, :]
        sid = sid_ref[0, 0, :]
        iota_p = lax.broadcasted_iota(jnp.int32, (TB, P), 1)
        iota_s = lax.broadcasted_iota(jnp.int32, (TB, S), 1)
        oh_p = (iota_p == pid[:, None]).astype(jnp.bfloat16)
        oh_s = (iota_s == sid[:, None]).astype(jnp.bfloat16)
        pp = jnp.dot(oh_p, ptab_ref[...], preferred_element_type=jnp.float32)
        ss = jnp.dot(oh_s, stab_ref[...], preferred_element_type=jnp.float32)
        o_ref[...] = w + pp + ss

    return pl.pallas_call(
        body,
        grid=(NB,),
        in_specs=[
            pl.BlockSpec((TB, DW), lambda i: (i, 0)),
            pl.BlockSpec((1, 1, TB), lambda i: (i, 0, 0)),
            pl.BlockSpec((1, 1, TB), lambda i: (i, 0, 0)),
            pl.BlockSpec((P, D), lambda i: (0, 0)),
            pl.BlockSpec((S, D), lambda i: (0, 0)),
        ],
        out_specs=pl.BlockSpec((TB, D), lambda i: (i, 0)),
        out_shape=jax.ShapeDtypeStruct((N, D), jnp.float32),
    )


def kernel(cat_ids, position_ids, shape_ids, word_table, pos_table, shape_table):
    B, L = cat_ids.shape
    V, D = word_table.shape
    P = pos_table.shape[0]
    S = shape_table.shape[0]
    N = B * L
    DW = D // 2
    TB = 4096

    wtab16 = word_table.astype(jnp.bfloat16)
    wtab_pairs = lax.bitcast_convert_type(
        jnp.stack([wtab16[:, :DW], wtab16[:, DW:]], axis=-1), jnp.int32)

    cat_flat = cat_ids.reshape(N)
    wrows = _make_sc_gather(N, DW)(cat_flat, wtab_pairs)
    out = _make_tc_assemble(N, D, DW, P, S, TB)(
        wrows.reshape(N, DW),
        position_ids.reshape(N // TB, 1, TB),
        shape_ids.reshape(N // TB, 1, TB),
        pos_table.astype(jnp.bfloat16),
        shape_table.astype(jnp.bfloat16),
    )
    return out.reshape(B, L, D)
